# baseline (speedup 1.0000x reference)
import functools

import jax
import jax.numpy as jnp
from jax.experimental import pallas as pl
from jax.experimental.pallas import tpu as pltpu

_K = 1024
_D = 64
_B = 32
_T = 1024
_TBLK = 1024
_NT = _T // _TBLK


def _vq_block(x_ref, z_ref, cb_ref,
              enc_ref, qst_ref, idx_ref, cnt_ref,
              kl_ref, el_ref, loss_ref, perp_ref, cbout_ref, cc_ref):
    b = pl.program_id(0)
    t = pl.program_id(1)
    first = jnp.logical_and(b == 0, t == 0)

    x = x_ref[0].T
    cb = cb_ref[...]
    zb = z_ref[...]

    @pl.when(first)
    def _():
        cc_ref[...] = jnp.sum(cb * cb, axis=1, keepdims=True)
        cbout_ref[...] = cb

    xx = jnp.sum(x * x, axis=1, keepdims=True).T
    cc = cc_ref[...]
    prod = jax.lax.dot_general(cb, x, (((1,), (1,)), ((), ())),
                               preferred_element_type=jnp.float32)
    dist = (xx + cc) - 2.0 * prod

    minv = jnp.min(dist, axis=0, keepdims=True)
    iota_k = jax.lax.broadcasted_iota(jnp.int32, (_K, _TBLK), 0)
    idx_row = jnp.min(jnp.where(dist <= minv, iota_k, _K),
                      axis=0, keepdims=True)
    idx_ref[0] = idx_row

    idx_col = idx_row.reshape(_TBLK, 1)
    iota_t = jax.lax.broadcasted_iota(jnp.int32, (_TBLK, _K), 1)
    enc = (iota_t == idx_col).astype(jnp.float32)
    enc_ref[...] = enc

    q = jnp.dot(enc, cb, preferred_element_type=jnp.float32)
    qst_ref[...] = x + (q - x)

    ones_k = jnp.ones((1, _K), jnp.float32)
    ones_t = jnp.ones((1, _TBLK), jnp.float32)
    red_k = lambda a: jax.lax.dot_general(
        ones_k, a, (((1,), (0,)), ((), ())),
        preferred_element_type=jnp.float32)

    lm = 0.5 * (minv - dist)
    e = jnp.exp(lm)
    s = red_k(e)
    t1 = red_k(e * lm)
    ez = jnp.exp(zb)
    sz = red_k(ez)
    t3 = red_k(e * zb)
    klrow = (t1 - t3) / s - jnp.log(s) + jnp.log(sz)

    klb = jnp.sum(klrow).reshape(1, 1)
    elb = jnp.sum((q - x) ** 2).reshape(1, 1)
    cntb = jax.lax.dot_general(
        ones_t, enc, (((1,), (0,)), ((), ())),
        preferred_element_type=jnp.float32)

    @pl.when(first)
    def _():
        cnt_ref[...] = cntb
        kl_ref[...] = klb
        el_ref[...] = elb

    @pl.when(jnp.logical_not(first))
    def _():
        cnt_ref[...] = cnt_ref[...] + cntb
        kl_ref[...] = kl_ref[...] + klb
        el_ref[...] = el_ref[...] + elb

    last = jnp.logical_and(b == _B - 1, t == _NT - 1)

    @pl.when(last)
    def _():
        kl_total = kl_ref[...]
        el_total = el_ref[...]
        n_el = float(_B * _T * _D)
        loss_ref[...] = 0.01 * (el_total / n_el) + kl_total / float(_B)
        avg = cnt_ref[...] / float(_B * _T)
        perp_ref[...] = jnp.exp(
            -jnp.sum(avg * jnp.log(avg + 1e-10))).reshape(1, 1)


@functools.partial(jax.jit, static_argnames=())
def kernel(y, z, codebook):
    z2 = z.reshape(_B * _K, _T)

    nblk = _B * _NT
    out_shapes = (
        jax.ShapeDtypeStruct((_B * _T, _K), jnp.float32),
        jax.ShapeDtypeStruct((_B * _T, _D), jnp.float32),
        jax.ShapeDtypeStruct((nblk, 1, _TBLK), jnp.int32),
        jax.ShapeDtypeStruct((1, _K), jnp.float32),
        jax.ShapeDtypeStruct((1, 1), jnp.float32),
        jax.ShapeDtypeStruct((1, 1), jnp.float32),
        jax.ShapeDtypeStruct((1, 1), jnp.float32),
        jax.ShapeDtypeStruct((1, 1), jnp.float32),
        jax.ShapeDtypeStruct((_K, _D), jnp.float32),
    )

    grid = (_B, _NT)
    in_specs = [
        pl.BlockSpec((1, _D, _TBLK), lambda b, t: (b, 0, t)),
        pl.BlockSpec((_K, _TBLK), lambda b, t: (b, t)),
        pl.BlockSpec((_K, _D), lambda b, t: (0, 0)),
    ]
    out_specs = (
        pl.BlockSpec((_TBLK, _K), lambda b, t: (b * _NT + t, 0)),
        pl.BlockSpec((_TBLK, _D), lambda b, t: (b * _NT + t, 0)),
        pl.BlockSpec((1, 1, _TBLK), lambda b, t: (b * _NT + t, 0, 0)),
        pl.BlockSpec((1, _K), lambda b, t: (0, 0)),
        pl.BlockSpec((1, 1), lambda b, t: (0, 0)),
        pl.BlockSpec((1, 1), lambda b, t: (0, 0)),
        pl.BlockSpec((1, 1), lambda b, t: (0, 0)),
        pl.BlockSpec((1, 1), lambda b, t: (0, 0)),
        pl.BlockSpec((_K, _D), lambda b, t: (0, 0)),
    )

    enc, qst, idx, _cnt, _kl, _el, loss, perp, cb_out = pl.pallas_call(
        _vq_block,
        grid=grid,
        in_specs=in_specs,
        out_specs=out_specs,
        out_shape=out_shapes,
        scratch_shapes=[pltpu.VMEM((_K, 1), jnp.float32)],
    )(y, z2, codebook)

    return (loss[0, 0],
            qst.reshape(_B, _T, _D),
            perp[0, 0],
            enc,
            cb_out,
            idx.reshape(_B * _T))


# speedup vs baseline: 1.0614x; 1.0614x over previous
import functools

import jax
import jax.numpy as jnp
from jax.experimental import pallas as pl
from jax.experimental.pallas import tpu as pltpu

_K = 1024
_D = 64
_B = 32
_T = 1024
_BB = 2
_NB = _B // _BB


def _vq_block(x_ref, z_ref, cb_ref,
              enc_ref, qst_ref, idx_ref, cnt_ref,
              kl_ref, el_ref, loss_ref, perp_ref, cbout_ref, cc_ref):
    i = pl.program_id(0)
    first = i == 0

    cb = cb_ref[...]

    @pl.when(first)
    def _():
        cc_ref[...] = jnp.sum(cb * cb, axis=1, keepdims=True)
        cbout_ref[...] = cb

    cc = cc_ref[...]
    ones_k = jnp.ones((1, _K), jnp.float32)
    ones_t = jnp.ones((1, _T), jnp.float32)
    red_k = lambda a: jax.lax.dot_general(
        ones_k, a, (((1,), (0,)), ((), ())),
        preferred_element_type=jnp.float32)

    klb = jnp.zeros((1, 1), jnp.float32)
    elb = jnp.zeros((1, 1), jnp.float32)
    cntb = jnp.zeros((1, _K), jnp.float32)

    for h in range(_BB):
        x = x_ref[h].T
        zb = z_ref[pl.ds(h * _K, _K), :]

        xx = jnp.sum(x * x, axis=1, keepdims=True).T
        prod = jax.lax.dot_general(cb, x, (((1,), (1,)), ((), ())),
                                   preferred_element_type=jnp.float32)
        dist = (xx + cc) - 2.0 * prod

        minv = jnp.min(dist, axis=0, keepdims=True)
        iota_k = jax.lax.broadcasted_iota(jnp.int32, (_K, _T), 0)
        idx_row = jnp.min(jnp.where(dist <= minv, iota_k, _K),
                          axis=0, keepdims=True)
        idx_ref[:, h, :] = idx_row

        idx_col = idx_row.reshape(_T, 1)
        iota_t = jax.lax.broadcasted_iota(jnp.int32, (_T, _K), 1)
        enc = (iota_t == idx_col).astype(jnp.float32)
        enc_ref[pl.ds(h * _T, _T), :] = enc

        q = jnp.dot(enc, cb, preferred_element_type=jnp.float32)
        qst_ref[pl.ds(h * _T, _T), :] = x + (q - x)

        lm = 0.5 * (minv - dist)
        e = jnp.exp(lm)
        s = red_k(e)
        t1 = red_k(e * lm)
        ez = jnp.exp(zb)
        sz = red_k(ez)
        t3 = red_k(e * zb)
        klrow = (t1 - t3) / s - jnp.log(s) + jnp.log(sz)

        klb = klb + jnp.sum(klrow).reshape(1, 1)
        elb = elb + jnp.sum((q - x) ** 2).reshape(1, 1)
        cntb = cntb + jax.lax.dot_general(
            ones_t, enc, (((1,), (0,)), ((), ())),
            preferred_element_type=jnp.float32)

    @pl.when(first)
    def _():
        cnt_ref[...] = cntb
        kl_ref[...] = klb
        el_ref[...] = elb

    @pl.when(jnp.logical_not(first))
    def _():
        cnt_ref[...] = cnt_ref[...] + cntb
        kl_ref[...] = kl_ref[...] + klb
        el_ref[...] = el_ref[...] + elb

    last = i == _NB - 1

    @pl.when(last)
    def _():
        kl_total = kl_ref[...]
        el_total = el_ref[...]
        n_el = float(_B * _T * _D)
        loss_ref[...] = 0.01 * (el_total / n_el) + kl_total / float(_B)
        avg = cnt_ref[...] / float(_B * _T)
        perp_ref[...] = jnp.exp(
            -jnp.sum(avg * jnp.log(avg + 1e-10))).reshape(1, 1)


@functools.partial(jax.jit, static_argnames=())
def kernel(y, z, codebook):
    z2 = z.reshape(_B * _K, _T)

    out_shapes = (
        jax.ShapeDtypeStruct((_B * _T, _K), jnp.float32),
        jax.ShapeDtypeStruct((_B * _T, _D), jnp.float32),
        jax.ShapeDtypeStruct((_NB, _BB, _T), jnp.int32),
        jax.ShapeDtypeStruct((1, _K), jnp.float32),
        jax.ShapeDtypeStruct((1, 1), jnp.float32),
        jax.ShapeDtypeStruct((1, 1), jnp.float32),
        jax.ShapeDtypeStruct((1, 1), jnp.float32),
        jax.ShapeDtypeStruct((1, 1), jnp.float32),
        jax.ShapeDtypeStruct((_K, _D), jnp.float32),
    )

    grid = (_NB,)
    in_specs = [
        pl.BlockSpec((_BB, _D, _T), lambda i: (i, 0, 0)),
        pl.BlockSpec((_BB * _K, _T), lambda i: (i, 0)),
        pl.BlockSpec((_K, _D), lambda i: (0, 0)),
    ]
    out_specs = (
        pl.BlockSpec((_BB * _T, _K), lambda i: (i, 0)),
        pl.BlockSpec((_BB * _T, _D), lambda i: (i, 0)),
        pl.BlockSpec((1, _BB, _T), lambda i: (i, 0, 0)),
        pl.BlockSpec((1, _K), lambda i: (0, 0)),
        pl.BlockSpec((1, 1), lambda i: (0, 0)),
        pl.BlockSpec((1, 1), lambda i: (0, 0)),
        pl.BlockSpec((1, 1), lambda i: (0, 0)),
        pl.BlockSpec((1, 1), lambda i: (0, 0)),
        pl.BlockSpec((_K, _D), lambda i: (0, 0)),
    )

    enc, qst, idx, _cnt, _kl, _el, loss, perp, cb_out = pl.pallas_call(
        _vq_block,
        grid=grid,
        in_specs=in_specs,
        out_specs=out_specs,
        out_shape=out_shapes,
        scratch_shapes=[pltpu.VMEM((_K, 1), jnp.float32)],
    )(y, z2, codebook)

    return (loss[0, 0],
            qst.reshape(_B, _T, _D),
            perp[0, 0],
            enc,
            cb_out,
            idx.reshape(_B * _T))
